# baseline (speedup 1.0000x reference)
import functools

import jax
import jax.numpy as jnp
from jax import lax
from jax.experimental import pallas as pl
from jax.experimental.pallas import tpu as pltpu
from jax.experimental.pallas import tpu_sc as plsc

BATCH = 32
INPUT_LEN = 4096
POOLED = 512
EMB = 64


def _bitonic_fold_desc(s, idx):
    B, L = s.shape
    iota = lax.broadcasted_iota(jnp.int32, (B, L), 1)
    des_fold = (iota & (L // 2)) == 0
    k = 2
    while k <= L:
        j = k // 2
        while j >= 1:
            up = (iota & j) == 0
            if k < L:
                des = (iota & k) == 0
            elif j == L // 2:
                des = jnp.full((B, L), True)
            else:
                des = des_fold
            s_p = jnp.where(up, jnp.roll(s, -j, axis=1), jnp.roll(s, j, axis=1))
            i_p = jnp.where(up, jnp.roll(idx, -j, axis=1), jnp.roll(idx, j, axis=1))
            gt = (s > s_p) | ((s == s_p) & (idx < i_p))
            keep = gt == (up == des)
            s = jnp.where(keep, s, s_p)
            idx = jnp.where(keep, idx, i_p)
            j //= 2
        k *= 2
    return s, idx


def _phase1_body(s_ref, s3_ref, c1a_ref, c1b_ref, w1a_ref, w1b_ref,
                 c2a_ref, c2b_ref, w2a_ref, w2b_ref,
                 c3a_ref, c3b_ref, w3a_ref, w3b_ref):
    s = s_ref[...]
    outs = ((c1a_ref, c1b_ref, w1a_ref, w1b_ref),
            (c2a_ref, c2b_ref, w2a_ref, w2b_ref),
            (c3a_ref, c3b_ref, w3a_ref, w3b_ref))
    for rnd in range(3):
        B, L = s.shape
        half = L // 2
        idx = lax.broadcasted_iota(jnp.int32, (B, L), 1)
        ss, ii = _bitonic_fold_desc(s, idx)
        sl = ss[:, :half]
        sr = ss[:, half:]
        ca = ii[:, :half]
        cb = ii[:, half:]
        xl = jnp.power(2.0, sl)
        xr = jnp.power(2.0, sr)
        m = jnp.maximum(xl, xr)
        el = jnp.exp(xl - m)
        er = jnp.exp(xr - m)
        tot = el + er
        wl = el / tot
        wr = er / tot
        s = sl * wl + sr * wr
        ca_ref, cb_ref, wa_ref, wb_ref = outs[rnd]
        ca_ref[...] = ca
        cb_ref[...] = cb
        wa_ref[...] = wl
        wb_ref[...] = wr
    s3_ref[...] = s


def _phase1(scores2d):
    f32 = jnp.float32
    i32 = jnp.int32
    nb = scores2d.shape[0]
    out_shapes = [
        jax.ShapeDtypeStruct((nb, POOLED), f32),
        jax.ShapeDtypeStruct((nb, 2048), i32),
        jax.ShapeDtypeStruct((nb, 2048), i32),
        jax.ShapeDtypeStruct((nb, 2048), f32),
        jax.ShapeDtypeStruct((nb, 2048), f32),
        jax.ShapeDtypeStruct((nb, 1024), i32),
        jax.ShapeDtypeStruct((nb, 1024), i32),
        jax.ShapeDtypeStruct((nb, 1024), f32),
        jax.ShapeDtypeStruct((nb, 1024), f32),
        jax.ShapeDtypeStruct((nb, POOLED), i32),
        jax.ShapeDtypeStruct((nb, POOLED), i32),
        jax.ShapeDtypeStruct((nb, POOLED), f32),
        jax.ShapeDtypeStruct((nb, POOLED), f32),
    ]
    return pl.pallas_call(
        _phase1_body,
        out_shape=out_shapes,
    )(scores2d)


def _take(a, i):
    B = a.shape[0]
    flat = i.reshape(B, -1)
    out = jnp.take_along_axis(a, flat, axis=1)
    return out.reshape(i.shape)


def _phase2_jax(embs, p1):
    (s3, c1a, c1b, w1a, w1b, c2a, c2b, w2a, w2b, c3a, c3b, w3a, w3b) = p1
    p2 = jnp.stack([c3a, c3b], axis=-1)
    v2 = jnp.stack([w3a, w3b], axis=-1)
    p1pos = jnp.stack([_take(c2a, p2), _take(c2b, p2)], axis=-1)
    v1 = v2[..., None] * jnp.stack([_take(w2a, p2), _take(w2b, p2)], axis=-1)
    p0 = jnp.stack([_take(c1a, p1pos), _take(c1b, p1pos)], axis=-1)
    v0 = v1[..., None] * jnp.stack([_take(w1a, p1pos), _take(w1b, p1pos)], axis=-1)
    B = embs.shape[0]
    idx = p0.reshape(B, -1)
    w = v0.reshape(B, -1)
    rows = jnp.take_along_axis(embs, idx[:, :, None], axis=1)
    rows = rows.reshape(B, POOLED, 8, EMB)
    w = w.reshape(B, POOLED, 8)
    out = (rows * w[..., None]).sum(axis=2)
    return out, s3


def _sc_body(emb_hbm, c1a, c1b, w1a, w1b, c2a, c2b, w2a, w2b,
             c3a, c3b, w3a, w3b, out_hbm,
             c1a_v, c1b_v, w1a_v, w1b_v, c2a_v, c2b_v, w2a_v, w2b_v,
             c3a_v, c3b_v, w3a_v, w3b_v, idx_buf, wgt_buf, rows_a, rows_b,
             out_stage, sem, sem2):
    b = lax.axis_index("s")
    h = lax.axis_index("c")
    pltpu.sync_copy(c1a.at[b], c1a_v)
    pltpu.sync_copy(c1b.at[b], c1b_v)
    pltpu.sync_copy(w1a.at[b], w1a_v)
    pltpu.sync_copy(w1b.at[b], w1b_v)
    pltpu.sync_copy(c2a.at[b], c2a_v)
    pltpu.sync_copy(c2b.at[b], c2b_v)
    pltpu.sync_copy(w2a.at[b], w2a_v)
    pltpu.sync_copy(w2b.at[b], w2b_v)
    pltpu.sync_copy(c3a.at[b], c3a_v)
    pltpu.sync_copy(c3b.at[b], c3b_v)
    pltpu.sync_copy(w3a.at[b], w3a_v)
    pltpu.sync_copy(w3b.at[b], w3b_v)

    lanes = lax.iota(jnp.int32, 16)
    boff = b * INPUT_LEN

    def chain_body(t, _):
        t16 = h * 256 + t * 16
        a3 = c3a_v[pl.ds(t16, 16)]
        b3 = c3b_v[pl.ds(t16, 16)]
        v0 = w3a_v[pl.ds(t16, 16)]
        v1 = w3b_v[pl.ds(t16, 16)]
        tsplat = jnp.full((16,), t, jnp.int32)
        for xsel in range(2):
            x = a3 if xsel == 0 else b3
            v = v0 if xsel == 0 else v1
            a2 = plsc.load_gather(c2a_v, [x])
            b2 = plsc.load_gather(c2b_v, [x])
            u0 = plsc.load_gather(w2a_v, [x]) * v
            u1 = plsc.load_gather(w2b_v, [x]) * v
            for ysel in range(2):
                y = a2 if ysel == 0 else b2
                u = u0 if ysel == 0 else u1
                d0 = plsc.load_gather(c1a_v, [y])
                d1 = plsc.load_gather(c1b_v, [y])
                f0 = plsc.load_gather(w1a_v, [y]) * u
                f1 = plsc.load_gather(w1b_v, [y]) * u
                for zsel in range(2):
                    k = xsel * 4 + ysel * 2 + zsel
                    cidx = d0 if zsel == 0 else d1
                    wz = f0 if zsel == 0 else f1
                    pos = lanes * 8 + k
                    plsc.store_scatter(idx_buf, [tsplat, pos], cidx + boff)
                    plsc.store_scatter(wgt_buf, [tsplat, pos], wz)
        return _

    lax.fori_loop(0, 16, chain_body, None)

    def accum(c, rows):
        csplat = jnp.full((16,), c, jnp.int32)
        for m in range(16):
            wb = [plsc.load_gather(
                      wgt_buf, [csplat, jnp.full((16,), m * 8 + k, jnp.int32)])
                  for k in range(8)]
            for d in range(4):
                acc = wb[0] * rows[m * 8, pl.ds(d * 16, 16)]
                for k in range(1, 8):
                    acc = acc + wb[k] * rows[m * 8 + k, pl.ds(d * 16, 16)]
                out_stage[c * 16 + m, pl.ds(d * 16, 16)] = acc

    pltpu.async_copy(emb_hbm.at[idx_buf.at[0]], rows_a, sem)

    def gather_body(g, _):
        c0 = 2 * g
        c1 = 2 * g + 1
        pltpu.async_copy(emb_hbm.at[idx_buf.at[c1]], rows_b, sem2)
        pltpu.make_async_copy(emb_hbm.at[idx_buf.at[c0]], rows_a, sem).wait()
        accum(c0, rows_a)

        @pl.when(g < 7)
        def _issue_next_even():
            pltpu.async_copy(emb_hbm.at[idx_buf.at[c0 + 2]], rows_a, sem)

        pltpu.make_async_copy(emb_hbm.at[idx_buf.at[c1]], rows_b, sem2).wait()
        accum(c1, rows_b)
        return _

    lax.fori_loop(0, 8, gather_body, None)
    pltpu.sync_copy(out_stage, out_hbm.at[pl.ds(b * POOLED + h * 256, 256)])


@jax.jit
def _phase2_sc(emb_flat, c1a, c1b, w1a, w1b, c2a, c2b, w2a, w2b,
               c3a, c3b, w3a, w3b):
    f32 = jnp.float32
    i32 = jnp.int32
    mesh = plsc.VectorSubcoreMesh(core_axis_name="c", subcore_axis_name="s")
    kern = functools.partial(
        pl.kernel,
        mesh=mesh,
        compiler_params=pltpu.CompilerParams(
            needs_layout_passes=False, use_tc_tiling_on_sc=False),
        out_type=jax.ShapeDtypeStruct((16 * POOLED, EMB), f32),
        scratch_types=[
            pltpu.VMEM((2048,), i32), pltpu.VMEM((2048,), i32),
            pltpu.VMEM((2048,), f32), pltpu.VMEM((2048,), f32),
            pltpu.VMEM((1024,), i32), pltpu.VMEM((1024,), i32),
            pltpu.VMEM((1024,), f32), pltpu.VMEM((1024,), f32),
            pltpu.VMEM((POOLED,), i32), pltpu.VMEM((POOLED,), i32),
            pltpu.VMEM((POOLED,), f32), pltpu.VMEM((POOLED,), f32),
            pltpu.VMEM((16, 128), i32), pltpu.VMEM((16, 128), f32),
            pltpu.VMEM((128, EMB), f32), pltpu.VMEM((128, EMB), f32),
            pltpu.VMEM((256, EMB), f32),
            pltpu.SemaphoreType.DMA, pltpu.SemaphoreType.DMA,
        ],
    )(_sc_body)
    return kern(emb_flat, c1a, c1b, w1a, w1b, c2a, c2b, w2a, w2b,
                c3a, c3b, w3a, w3b)


@jax.jit
def _impl(embs, scores):
    scores2d = scores[:, :, 0]
    emb_parts = []
    score_parts = []
    for c in range(2):
        lo = c * 16
        p1 = _phase1(scores2d[lo:lo + 16])
        score_parts.append(p1[0])
        emb_flat = embs[lo:lo + 16].reshape(16 * INPUT_LEN, EMB)
        out_flat = _phase2_sc(emb_flat, *p1[1:])
        emb_parts.append(out_flat.reshape(16, POOLED, EMB))
    return (jnp.concatenate(emb_parts, axis=0),
            jnp.concatenate(score_parts, axis=0))


def kernel(embs, scores):
    return _impl(embs, scores)


# speedup vs baseline: 1.2207x; 1.2207x over previous
import functools

import jax
import jax.numpy as jnp
from jax import lax
from jax.experimental import pallas as pl
from jax.experimental.pallas import tpu as pltpu
from jax.experimental.pallas import tpu_sc as plsc

BATCH = 32
INPUT_LEN = 4096
POOLED = 512
EMB = 64


def _bitonic_fold_desc(s, idx):
    B, L = s.shape
    iota = lax.broadcasted_iota(jnp.int32, (B, L), 1)
    des_fold = (iota & (L // 2)) == 0
    k = 2
    while k <= L:
        j = k // 2
        while j >= 1:
            up = (iota & j) == 0
            if k < L:
                des = (iota & k) == 0
            elif j == L // 2:
                des = jnp.full((B, L), True)
            else:
                des = des_fold
            s_p = jnp.where(up, pltpu.roll(s, L - j, 1), pltpu.roll(s, j, 1))
            i_p = jnp.where(up, pltpu.roll(idx, L - j, 1), pltpu.roll(idx, j, 1))
            gt = (s > s_p) | ((s == s_p) & (idx < i_p))
            keep = gt == (up == des)
            s = jnp.where(keep, s, s_p)
            idx = jnp.where(keep, idx, i_p)
            j //= 2
        k *= 2
    return s, idx


def _phase1_body(s_ref, s3_ref, c1a_ref, c1b_ref, w1a_ref, w1b_ref,
                 c2a_ref, c2b_ref, w2a_ref, w2b_ref,
                 c3a_ref, c3b_ref, w3a_ref, w3b_ref):
    s = s_ref[...]
    outs = ((c1a_ref, c1b_ref, w1a_ref, w1b_ref),
            (c2a_ref, c2b_ref, w2a_ref, w2b_ref),
            (c3a_ref, c3b_ref, w3a_ref, w3b_ref))
    for rnd in range(3):
        B, L = s.shape
        half = L // 2
        idx = lax.broadcasted_iota(jnp.int32, (B, L), 1)
        ss, ii = _bitonic_fold_desc(s, idx)
        sl = ss[:, :half]
        sr = ss[:, half:]
        ca = ii[:, :half]
        cb = ii[:, half:]
        xl = jnp.power(2.0, sl)
        xr = jnp.power(2.0, sr)
        m = jnp.maximum(xl, xr)
        el = jnp.exp(xl - m)
        er = jnp.exp(xr - m)
        tot = el + er
        wl = el / tot
        wr = er / tot
        s = sl * wl + sr * wr
        ca_ref, cb_ref, wa_ref, wb_ref = outs[rnd]
        ca_ref[...] = ca
        cb_ref[...] = cb
        wa_ref[...] = wl
        wb_ref[...] = wr
    s3_ref[...] = s


def _phase1(scores2d):
    f32 = jnp.float32
    i32 = jnp.int32
    nb = scores2d.shape[0]
    out_shapes = [
        jax.ShapeDtypeStruct((nb, POOLED), f32),
        jax.ShapeDtypeStruct((nb, 2048), i32),
        jax.ShapeDtypeStruct((nb, 2048), i32),
        jax.ShapeDtypeStruct((nb, 2048), f32),
        jax.ShapeDtypeStruct((nb, 2048), f32),
        jax.ShapeDtypeStruct((nb, 1024), i32),
        jax.ShapeDtypeStruct((nb, 1024), i32),
        jax.ShapeDtypeStruct((nb, 1024), f32),
        jax.ShapeDtypeStruct((nb, 1024), f32),
        jax.ShapeDtypeStruct((nb, POOLED), i32),
        jax.ShapeDtypeStruct((nb, POOLED), i32),
        jax.ShapeDtypeStruct((nb, POOLED), f32),
        jax.ShapeDtypeStruct((nb, POOLED), f32),
    ]
    return pl.pallas_call(
        _phase1_body,
        out_shape=out_shapes,
    )(scores2d)


def _take(a, i):
    B = a.shape[0]
    flat = i.reshape(B, -1)
    out = jnp.take_along_axis(a, flat, axis=1)
    return out.reshape(i.shape)


def _phase2_jax(embs, p1):
    (s3, c1a, c1b, w1a, w1b, c2a, c2b, w2a, w2b, c3a, c3b, w3a, w3b) = p1
    p2 = jnp.stack([c3a, c3b], axis=-1)
    v2 = jnp.stack([w3a, w3b], axis=-1)
    p1pos = jnp.stack([_take(c2a, p2), _take(c2b, p2)], axis=-1)
    v1 = v2[..., None] * jnp.stack([_take(w2a, p2), _take(w2b, p2)], axis=-1)
    p0 = jnp.stack([_take(c1a, p1pos), _take(c1b, p1pos)], axis=-1)
    v0 = v1[..., None] * jnp.stack([_take(w1a, p1pos), _take(w1b, p1pos)], axis=-1)
    B = embs.shape[0]
    idx = p0.reshape(B, -1)
    w = v0.reshape(B, -1)
    rows = jnp.take_along_axis(embs, idx[:, :, None], axis=1)
    rows = rows.reshape(B, POOLED, 8, EMB)
    w = w.reshape(B, POOLED, 8)
    out = (rows * w[..., None]).sum(axis=2)
    return out, s3


def _sc_body(emb_hbm, c1a, c1b, w1a, w1b, c2a, c2b, w2a, w2b,
             c3a, c3b, w3a, w3b, out_hbm,
             c1a_v, c1b_v, w1a_v, w1b_v, c2a_v, c2b_v, w2a_v, w2b_v,
             c3a_v, c3b_v, w3a_v, w3b_v, idx_buf, wgt_buf, rows_a, rows_b,
             out_stage, sem, sem2):
    b = lax.axis_index("s") * 2 + lax.axis_index("c")
    pltpu.sync_copy(c1a.at[b], c1a_v)
    pltpu.sync_copy(c1b.at[b], c1b_v)
    pltpu.sync_copy(w1a.at[b], w1a_v)
    pltpu.sync_copy(w1b.at[b], w1b_v)
    pltpu.sync_copy(c2a.at[b], c2a_v)
    pltpu.sync_copy(c2b.at[b], c2b_v)
    pltpu.sync_copy(w2a.at[b], w2a_v)
    pltpu.sync_copy(w2b.at[b], w2b_v)
    pltpu.sync_copy(c3a.at[b], c3a_v)
    pltpu.sync_copy(c3b.at[b], c3b_v)
    pltpu.sync_copy(w3a.at[b], w3a_v)
    pltpu.sync_copy(w3b.at[b], w3b_v)

    lanes = lax.iota(jnp.int32, 16)
    boff = b * INPUT_LEN

    def chain_body(t, _):
        t16 = t * 16
        a3 = c3a_v[pl.ds(t16, 16)]
        b3 = c3b_v[pl.ds(t16, 16)]
        v0 = w3a_v[pl.ds(t16, 16)]
        v1 = w3b_v[pl.ds(t16, 16)]
        tsplat = jnp.full((16,), t, jnp.int32)
        for xsel in range(2):
            x = a3 if xsel == 0 else b3
            v = v0 if xsel == 0 else v1
            a2 = plsc.load_gather(c2a_v, [x])
            b2 = plsc.load_gather(c2b_v, [x])
            u0 = plsc.load_gather(w2a_v, [x]) * v
            u1 = plsc.load_gather(w2b_v, [x]) * v
            for ysel in range(2):
                y = a2 if ysel == 0 else b2
                u = u0 if ysel == 0 else u1
                d0 = plsc.load_gather(c1a_v, [y])
                d1 = plsc.load_gather(c1b_v, [y])
                f0 = plsc.load_gather(w1a_v, [y]) * u
                f1 = plsc.load_gather(w1b_v, [y]) * u
                for zsel in range(2):
                    k = xsel * 4 + ysel * 2 + zsel
                    cidx = d0 if zsel == 0 else d1
                    wz = f0 if zsel == 0 else f1
                    pos = lanes * 8 + k
                    plsc.store_scatter(idx_buf, [tsplat, pos], cidx + boff)
                    plsc.store_scatter(wgt_buf, [tsplat, pos], wz)
        return _

    lax.fori_loop(0, 32, chain_body, None)

    def accum(c, rows):
        csplat = jnp.full((16,), c, jnp.int32)
        for m in range(16):
            wb = [plsc.load_gather(
                      wgt_buf, [csplat, jnp.full((16,), m * 8 + k, jnp.int32)])
                  for k in range(8)]
            for d in range(4):
                acc = wb[0] * rows[m * 8, pl.ds(d * 16, 16)]
                for k in range(1, 8):
                    acc = acc + wb[k] * rows[m * 8 + k, pl.ds(d * 16, 16)]
                out_stage[c * 16 + m, pl.ds(d * 16, 16)] = acc

    pltpu.async_copy(emb_hbm.at[idx_buf.at[0]], rows_a, sem)

    def gather_body(g, _):
        c0 = 2 * g
        c1 = 2 * g + 1
        pltpu.async_copy(emb_hbm.at[idx_buf.at[c1]], rows_b, sem2)
        pltpu.make_async_copy(emb_hbm.at[idx_buf.at[c0]], rows_a, sem).wait()
        accum(c0, rows_a)

        @pl.when(g < 15)
        def _issue_next_even():
            pltpu.async_copy(emb_hbm.at[idx_buf.at[c0 + 2]], rows_a, sem)

        pltpu.make_async_copy(emb_hbm.at[idx_buf.at[c1]], rows_b, sem2).wait()
        accum(c1, rows_b)
        return _

    lax.fori_loop(0, 16, gather_body, None)
    pltpu.sync_copy(out_stage, out_hbm.at[pl.ds(b * POOLED, POOLED)])


@jax.jit
def _phase2_sc(emb_flat, c1a, c1b, w1a, w1b, c2a, c2b, w2a, w2b,
               c3a, c3b, w3a, w3b):
    f32 = jnp.float32
    i32 = jnp.int32
    mesh = plsc.VectorSubcoreMesh(core_axis_name="c", subcore_axis_name="s")
    kern = functools.partial(
        pl.kernel,
        mesh=mesh,
        compiler_params=pltpu.CompilerParams(
            needs_layout_passes=False, use_tc_tiling_on_sc=False),
        out_type=jax.ShapeDtypeStruct((BATCH * POOLED, EMB), f32),
        scratch_types=[
            pltpu.VMEM((2048,), i32), pltpu.VMEM((2048,), i32),
            pltpu.VMEM((2048,), f32), pltpu.VMEM((2048,), f32),
            pltpu.VMEM((1024,), i32), pltpu.VMEM((1024,), i32),
            pltpu.VMEM((1024,), f32), pltpu.VMEM((1024,), f32),
            pltpu.VMEM((POOLED,), i32), pltpu.VMEM((POOLED,), i32),
            pltpu.VMEM((POOLED,), f32), pltpu.VMEM((POOLED,), f32),
            pltpu.VMEM((32, 128), i32), pltpu.VMEM((32, 128), f32),
            pltpu.VMEM((128, EMB), f32), pltpu.VMEM((128, EMB), f32),
            pltpu.VMEM((POOLED, EMB), f32),
            pltpu.SemaphoreType.DMA, pltpu.SemaphoreType.DMA,
        ],
    )(_sc_body)
    return kern(emb_flat, c1a, c1b, w1a, w1b, c2a, c2b, w2a, w2b,
                c3a, c3b, w3a, w3b)


@jax.jit
def _impl(embs, scores):
    scores2d = scores[:, :, 0]
    p1 = _phase1(scores2d)
    emb_flat = embs.reshape(BATCH * INPUT_LEN, EMB)
    out_flat = _phase2_sc(emb_flat, *p1[1:])
    return (out_flat.reshape(BATCH, POOLED, EMB), p1[0])


def kernel(embs, scores):
    return _impl(embs, scores)


# speedup vs baseline: 1.2597x; 1.0319x over previous
import functools

import jax
import jax.numpy as jnp
from jax import lax
from jax.experimental import pallas as pl
from jax.experimental.pallas import tpu as pltpu
from jax.experimental.pallas import tpu_sc as plsc

BATCH = 32
INPUT_LEN = 4096
POOLED = 512
EMB = 64


def _bitonic_fold_desc(s, idx):
    B, L = s.shape
    iota = lax.broadcasted_iota(jnp.int32, (B, L), 1)
    des_fold = (iota & (L // 2)) == 0
    k = 2
    while k <= L:
        j = k // 2
        while j >= 1:
            up = (iota & j) == 0
            if k < L:
                des = (iota & k) == 0
            elif j == L // 2:
                des = jnp.full((B, L), True)
            else:
                des = des_fold
            s_p = jnp.where(up, jnp.roll(s, -j, axis=1), jnp.roll(s, j, axis=1))
            i_p = jnp.where(up, jnp.roll(idx, -j, axis=1), jnp.roll(idx, j, axis=1))
            gt = (s > s_p) | ((s == s_p) & (idx < i_p))
            keep = gt == (up == des)
            s = jnp.where(keep, s, s_p)
            idx = jnp.where(keep, idx, i_p)
            j //= 2
        k *= 2
    return s, idx


def _phase1_body(s_ref, s3_ref, c1a_ref, c1b_ref, w1a_ref, w1b_ref,
                 c2a_ref, c2b_ref, w2a_ref, w2b_ref,
                 c3a_ref, c3b_ref, w3a_ref, w3b_ref):
    s = s_ref[...]
    outs = ((c1a_ref, c1b_ref, w1a_ref, w1b_ref),
            (c2a_ref, c2b_ref, w2a_ref, w2b_ref),
            (c3a_ref, c3b_ref, w3a_ref, w3b_ref))
    for rnd in range(3):
        B, L = s.shape
        half = L // 2
        idx = lax.broadcasted_iota(jnp.int32, (B, L), 1)
        ss, ii = _bitonic_fold_desc(s, idx)
        sl = ss[:, :half]
        sr = ss[:, half:]
        ca = ii[:, :half]
        cb = ii[:, half:]
        xl = jnp.power(2.0, sl)
        xr = jnp.power(2.0, sr)
        m = jnp.maximum(xl, xr)
        el = jnp.exp(xl - m)
        er = jnp.exp(xr - m)
        tot = el + er
        wl = el / tot
        wr = er / tot
        s = sl * wl + sr * wr
        ca_ref, cb_ref, wa_ref, wb_ref = outs[rnd]
        ca_ref[...] = ca
        cb_ref[...] = cb
        wa_ref[...] = wl
        wb_ref[...] = wr
    s3_ref[...] = s


def _phase1(scores2d):
    f32 = jnp.float32
    i32 = jnp.int32
    nb = scores2d.shape[0]
    out_shapes = [
        jax.ShapeDtypeStruct((nb, POOLED), f32),
        jax.ShapeDtypeStruct((nb, 2048), i32),
        jax.ShapeDtypeStruct((nb, 2048), i32),
        jax.ShapeDtypeStruct((nb, 2048), f32),
        jax.ShapeDtypeStruct((nb, 2048), f32),
        jax.ShapeDtypeStruct((nb, 1024), i32),
        jax.ShapeDtypeStruct((nb, 1024), i32),
        jax.ShapeDtypeStruct((nb, 1024), f32),
        jax.ShapeDtypeStruct((nb, 1024), f32),
        jax.ShapeDtypeStruct((nb, POOLED), i32),
        jax.ShapeDtypeStruct((nb, POOLED), i32),
        jax.ShapeDtypeStruct((nb, POOLED), f32),
        jax.ShapeDtypeStruct((nb, POOLED), f32),
    ]
    return pl.pallas_call(
        _phase1_body,
        out_shape=out_shapes,
    )(scores2d)


def _sc_body(emb_hbm, c1a, c1b, w1a, w1b, c2a, c2b, w2a, w2b,
             c3a, c3b, w3a, w3b, out_hbm,
             c1a_v, c1b_v, w1a_v, w1b_v, c2a_v, c2b_v, w2a_v, w2b_v,
             c3a_v, c3b_v, w3a_v, w3b_v, idx_buf, wgt_buf, rows_a,
             out_stage, sem):
    b = lax.axis_index("s") * 2 + lax.axis_index("c")
    pltpu.sync_copy(c1a.at[b], c1a_v)
    pltpu.sync_copy(c1b.at[b], c1b_v)
    pltpu.sync_copy(w1a.at[b], w1a_v)
    pltpu.sync_copy(w1b.at[b], w1b_v)
    pltpu.sync_copy(c2a.at[b], c2a_v)
    pltpu.sync_copy(c2b.at[b], c2b_v)
    pltpu.sync_copy(w2a.at[b], w2a_v)
    pltpu.sync_copy(w2b.at[b], w2b_v)
    pltpu.sync_copy(c3a.at[b], c3a_v)
    pltpu.sync_copy(c3b.at[b], c3b_v)
    pltpu.sync_copy(w3a.at[b], w3a_v)
    pltpu.sync_copy(w3b.at[b], w3b_v)

    lanes = lax.iota(jnp.int32, 16)
    boff = b * INPUT_LEN

    def chain_body(t, _):
        t16 = t * 16
        a3 = c3a_v[pl.ds(t16, 16)]
        b3 = c3b_v[pl.ds(t16, 16)]
        v0 = w3a_v[pl.ds(t16, 16)]
        v1 = w3b_v[pl.ds(t16, 16)]
        tsplat = jnp.full((16,), t, jnp.int32)
        for xsel in range(2):
            x = a3 if xsel == 0 else b3
            v = v0 if xsel == 0 else v1
            a2 = plsc.load_gather(c2a_v, [x])
            b2 = plsc.load_gather(c2b_v, [x])
            u0 = plsc.load_gather(w2a_v, [x]) * v
            u1 = plsc.load_gather(w2b_v, [x]) * v
            for ysel in range(2):
                y = a2 if ysel == 0 else b2
                u = u0 if ysel == 0 else u1
                d0 = plsc.load_gather(c1a_v, [y])
                d1 = plsc.load_gather(c1b_v, [y])
                f0 = plsc.load_gather(w1a_v, [y]) * u
                f1 = plsc.load_gather(w1b_v, [y]) * u
                for zsel in range(2):
                    k = xsel * 4 + ysel * 2 + zsel
                    cidx = d0 if zsel == 0 else d1
                    wz = f0 if zsel == 0 else f1
                    pos = lanes * 8 + k
                    plsc.store_scatter(idx_buf, [tsplat, pos], cidx + boff)
                    plsc.store_scatter(wgt_buf, [tsplat, pos], wz)
        return _

    lax.fori_loop(0, 32, chain_body, None)

    def accum(c, rows):
        csplat = jnp.full((16,), c, jnp.int32)
        for m in range(16):
            wb = [plsc.load_gather(
                      wgt_buf, [csplat, jnp.full((16,), m * 8 + k, jnp.int32)])
                  for k in range(8)]
            for d in range(4):
                acc = wb[0] * rows[m * 8, pl.ds(d * 16, 16)]
                for k in range(1, 8):
                    acc = acc + wb[k] * rows[m * 8 + k, pl.ds(d * 16, 16)]
                out_stage[c * 16 + m, pl.ds(d * 16, 16)] = acc

    def gather_body(c, _):
        pltpu.async_copy(emb_hbm.at[idx_buf.at[c]], rows_a, sem).wait()
        accum(c, rows_a)
        return _

    lax.fori_loop(0, 32, gather_body, None)
    pltpu.sync_copy(out_stage, out_hbm.at[pl.ds(b * POOLED, POOLED)])


@jax.jit
def _phase2_sc(emb_flat, c1a, c1b, w1a, w1b, c2a, c2b, w2a, w2b,
               c3a, c3b, w3a, w3b):
    f32 = jnp.float32
    i32 = jnp.int32
    mesh = plsc.VectorSubcoreMesh(core_axis_name="c", subcore_axis_name="s")
    kern = functools.partial(
        pl.kernel,
        mesh=mesh,
        compiler_params=pltpu.CompilerParams(
            needs_layout_passes=False, use_tc_tiling_on_sc=False),
        out_type=jax.ShapeDtypeStruct((BATCH * POOLED, EMB), f32),
        scratch_types=[
            pltpu.VMEM((2048,), i32), pltpu.VMEM((2048,), i32),
            pltpu.VMEM((2048,), f32), pltpu.VMEM((2048,), f32),
            pltpu.VMEM((1024,), i32), pltpu.VMEM((1024,), i32),
            pltpu.VMEM((1024,), f32), pltpu.VMEM((1024,), f32),
            pltpu.VMEM((POOLED,), i32), pltpu.VMEM((POOLED,), i32),
            pltpu.VMEM((POOLED,), f32), pltpu.VMEM((POOLED,), f32),
            pltpu.VMEM((32, 128), i32), pltpu.VMEM((32, 128), f32),
            pltpu.VMEM((128, EMB), f32),
            pltpu.VMEM((POOLED, EMB), f32),
            pltpu.SemaphoreType.DMA,
        ],
    )(_sc_body)
    return kern(emb_flat, c1a, c1b, w1a, w1b, c2a, c2b, w2a, w2b,
                c3a, c3b, w3a, w3b)


@jax.jit
def _impl(embs, scores):
    scores2d = scores[:, :, 0]
    p1 = _phase1(scores2d)
    emb_flat = embs.reshape(BATCH * INPUT_LEN, EMB)
    out_flat = _phase2_sc(emb_flat, *p1[1:])
    return (out_flat.reshape(BATCH, POOLED, EMB), p1[0])


def kernel(embs, scores):
    return _impl(embs, scores)
